# baseline (speedup 1.0000x reference)
import functools

import jax
import jax.numpy as jnp
from jax import lax
from jax.experimental import pallas as pl
from jax.experimental.pallas import tpu as pltpu
from jax.experimental.pallas import tpu_sc as plsc

BLK = 256

SC_CORES = 2
SC_SUBCORES = 16
SC_WORKERS = SC_CORES * SC_SUBCORES


def _sc_row_gather(table, idx):
    n_rows, d = table.shape
    b = idx.shape[0]
    rows_per_w = b // SC_WORKERS
    mesh = plsc.VectorSubcoreMesh(core_axis_name="c", subcore_axis_name="s")

    @functools.partial(
        pl.kernel,
        mesh=mesh,
        out_type=jax.ShapeDtypeStruct((b, d), table.dtype),
        scratch_types=[
            pltpu.VMEM((rows_per_w,), jnp.int32),
            pltpu.VMEM((rows_per_w, d), table.dtype),
            pltpu.SemaphoreType.DMA,
        ],
    )
    def gather_kernel(table_hbm, idx_hbm, out_hbm, idx_v, rows_v, sem):
        wid = lax.axis_index("s") * SC_CORES + lax.axis_index("c")
        base = wid * rows_per_w
        pltpu.sync_copy(idx_hbm.at[pl.ds(base, rows_per_w)], idx_v)
        pltpu.async_copy(table_hbm.at[idx_v], rows_v, sem).wait()
        pltpu.sync_copy(rows_v, out_hbm.at[pl.ds(base, rows_per_w)])

    return gather_kernel(table, idx)


def _mm_body(tile_ids, expert_sel, expert_cmp, xs_ref, w_ref, b_ref, es_ref,
             out_ref):
    i = pl.program_id(0)
    t_cur = tile_ids[i]
    t_prev = tile_ids[jnp.maximum(i - 1, 0)]
    first_visit = jnp.logical_or(i == 0, t_cur != t_prev)
    ecmp = expert_cmp[i]

    @pl.when(first_visit)
    def _():
        out_ref[...] = jnp.zeros_like(out_ref)

    @pl.when(ecmp >= 0)
    def _():
        mask = es_ref[0] == ecmp
        acc = jnp.dot(xs_ref[...].astype(jnp.bfloat16),
                      w_ref[0].astype(jnp.bfloat16),
                      preferred_element_type=jnp.float32)
        contrib = jnp.where(mask, acc + b_ref[0], 0.0)
        out_ref[...] += contrib


def _grouped_matmul(xs, w, b3, es3, tile_ids, expert_sel, expert_cmp):
    n, d = xs.shape
    num_items = tile_ids.shape[0]
    grid_spec = pltpu.PrefetchScalarGridSpec(
        num_scalar_prefetch=3,
        grid=(num_items,),
        in_specs=[
            pl.BlockSpec((BLK, d), lambda i, t, es, ec: (t[i], 0)),
            pl.BlockSpec((1, d, d), lambda i, t, es, ec: (es[i], 0, 0)),
            pl.BlockSpec((1, 1, d), lambda i, t, es, ec: (es[i], 0, 0)),
            pl.BlockSpec((1, BLK, 1), lambda i, t, es, ec: (t[i], 0, 0)),
        ],
        out_specs=pl.BlockSpec((BLK, d), lambda i, t, es, ec: (t[i], 0)),
    )
    return pl.pallas_call(
        _mm_body,
        grid_spec=grid_spec,
        out_shape=jax.ShapeDtypeStruct((n, d), jnp.float32),
        compiler_params=pltpu.CompilerParams(
            dimension_semantics=("arbitrary",)),
    )(tile_ids, expert_sel, expert_cmp, xs, w, b3, es3)


def kernel(x, gate_idx, W, b):
    n, d = x.shape
    e_total = W.shape[0]
    num_tiles = n // BLK
    num_items = num_tiles + e_total - 1

    e = gate_idx[:, 0].astype(jnp.int32)
    iota = jnp.arange(n, dtype=jnp.int32)
    e_sorted, perm = lax.sort_key_val(e, iota)
    inv_perm = jnp.zeros((n,), jnp.int32).at[perm].set(iota)

    counts = jnp.sum(
        (e[None, :] == jnp.arange(e_total, dtype=jnp.int32)[:, None]).astype(
            jnp.int32),
        axis=1)
    ends = jnp.cumsum(counts).astype(jnp.int32)
    starts = ends - counts

    pair = jnp.arange(num_tiles * e_total, dtype=jnp.int32)
    t_ids = pair // e_total
    e_ids = pair % e_total
    seg_start = starts[e_ids]
    seg_end = ends[e_ids]
    blk_start = t_ids * BLK
    valid = ((seg_start < blk_start + BLK) & (seg_end > blk_start)
             & (seg_end > seg_start))
    key = jnp.where(valid, pair, jnp.int32(num_tiles * e_total))
    key = jnp.sort(key)[:num_items]
    is_real = key < num_tiles * e_total
    tile_ids = jnp.where(is_real, key // e_total, num_tiles - 1)
    expert_cmp = jnp.where(is_real, key % e_total, -1)
    expert_sel = jnp.where(is_real, key % e_total, e_total - 1)

    xs = _sc_row_gather(x, perm)
    es3 = e_sorted.reshape(num_tiles, BLK, 1)
    b3 = b.reshape(e_total, 1, d)
    ys = _grouped_matmul(xs, W, b3, es3, tile_ids, expert_sel, expert_cmp)
    return _sc_row_gather(ys, inv_perm)


# speedup vs baseline: 1.2362x; 1.2362x over previous
import functools

import jax
import jax.numpy as jnp
from jax import lax
from jax.experimental import pallas as pl
from jax.experimental.pallas import tpu as pltpu
from jax.experimental.pallas import tpu_sc as plsc

BLK = 256

SC_CORES = 2
SC_SUBCORES = 16
SC_WORKERS = SC_CORES * SC_SUBCORES


def _sc_route_and_scatter(x, e_flat, n_expert):
    n, d = x.shape
    per_w = n // SC_WORKERS
    n_vec = per_w // 16
    mesh = plsc.VectorSubcoreMesh(core_axis_name="c", subcore_axis_name="s")

    @functools.partial(
        pl.kernel,
        mesh=mesh,
        compiler_params=pltpu.CompilerParams(needs_layout_passes=False),
        out_type=(
            jax.ShapeDtypeStruct((n, d), x.dtype),
            jax.ShapeDtypeStruct((n,), jnp.int32),
            jax.ShapeDtypeStruct((16,), jnp.int32),
            jax.ShapeDtypeStruct((16,), jnp.int32),
            jax.ShapeDtypeStruct((SC_WORKERS, 16), jnp.int32),
        ),
        scratch_types=[
            pltpu.VMEM((per_w,), jnp.int32),
            pltpu.VMEM((per_w,), jnp.int32),
            pltpu.VMEM((per_w,), jnp.int32),
            pltpu.VMEM((per_w, d), x.dtype),
            pltpu.VMEM((16,), jnp.int32),
            pltpu.VMEM((SC_WORKERS, 16), jnp.int32),
            pltpu.SemaphoreType.DMA,
            pltpu.SemaphoreType.DMA,
        ],
    )
    def route_kernel(x_hbm, e_hbm, xs_hbm, invp_hbm, cnt_hbm, st_hbm,
                     tbl_hbm, e_v, sib_v, dest_v, rows_v, stage_v, tbl_v,
                     sem_x, sem_sc):
        sid = lax.axis_index("s")
        cid = lax.axis_index("c")
        wid = sid * SC_CORES + cid
        base = wid * per_w
        lane = lax.iota(jnp.int32, 16)

        x_copy = pltpu.async_copy(x_hbm.at[pl.ds(base, per_w)], rows_v,
                                  sem_x)
        pltpu.sync_copy(e_hbm.at[pl.ds(base, per_w)], e_v)

        sib = sid * 2 + (1 - cid)
        pltpu.sync_copy(e_hbm.at[pl.ds(sib * per_w, per_w)], sib_v)
        for (src, row) in ((e_v, wid), (sib_v, sib)):
            vecs_h = [src[pl.ds(k * 16, 16)] for k in range(n_vec)]
            cnt_vec = jnp.zeros((16,), jnp.int32)
            for ex in range(n_expert):
                c = jnp.int32(0)
                for v in vecs_h:
                    c = c + jnp.sum((v == ex).astype(jnp.int32))
                cnt_vec = cnt_vec + jnp.where(lane == ex, c, 0)
            stage_v[...] = cnt_vec
            pltpu.sync_copy(stage_v, tbl_hbm.at[row])
        plsc.subcore_barrier()
        vecs = [e_v[pl.ds(k * 16, 16)] for k in range(n_vec)]

        pltpu.sync_copy(tbl_hbm, tbl_v)
        total_vec = jnp.zeros((16,), jnp.int32)
        before_vec = jnp.zeros((16,), jnp.int32)
        for r in range(SC_WORKERS):
            row = tbl_v[r]
            total_vec = total_vec + row
            before_vec = before_vec + row * jnp.where(
                jnp.int32(r) < wid, jnp.int32(1), jnp.int32(0))
        incl = jnp.cumsum(total_vec)
        starts_vec = incl - total_vec
        my_base_vec = starts_vec + before_vec

        @pl.when(wid == 0)
        def _():
            stage_v[...] = total_vec
            pltpu.sync_copy(stage_v, cnt_hbm)
            stage_v[...] = starts_vec
            pltpu.sync_copy(stage_v, st_hbm)

        bases = [
            jnp.sum(jnp.where(lane == ex, my_base_vec, 0))
            for ex in range(n_expert)
        ]
        for k in range(n_vec):
            v = vecs[k]
            dest = jnp.zeros((16,), jnp.int32)
            for ex in range(n_expert):
                m = (v == ex).astype(jnp.int32)
                pc = jnp.cumsum(m)
                dest = dest + m * (bases[ex] + pc - 1)
                bases[ex] = bases[ex] + jnp.sum(m)
            dest_v[pl.ds(k * 16, 16)] = dest

        x_copy.wait()
        pltpu.async_copy(rows_v, xs_hbm.at[dest_v], sem_sc).wait()
        pltpu.sync_copy(dest_v, invp_hbm.at[pl.ds(base, per_w)])

    xs, invp, counts16, starts16, _ = route_kernel(x, e_flat)
    return xs, invp, counts16, starts16


def _sc_row_gather(table, idx):
    n_rows, d = table.shape
    b = idx.shape[0]
    rows_per_w = b // SC_WORKERS
    mesh = plsc.VectorSubcoreMesh(core_axis_name="c", subcore_axis_name="s")

    @functools.partial(
        pl.kernel,
        mesh=mesh,
        out_type=jax.ShapeDtypeStruct((b, d), table.dtype),
        scratch_types=[
            pltpu.VMEM((rows_per_w,), jnp.int32),
            pltpu.VMEM((rows_per_w, d), table.dtype),
            pltpu.SemaphoreType.DMA,
        ],
    )
    def gather_kernel(table_hbm, idx_hbm, out_hbm, idx_v, rows_v, sem):
        wid = lax.axis_index("s") * SC_CORES + lax.axis_index("c")
        base = wid * rows_per_w
        pltpu.sync_copy(idx_hbm.at[pl.ds(base, rows_per_w)], idx_v)
        pltpu.async_copy(table_hbm.at[idx_v], rows_v, sem).wait()
        pltpu.sync_copy(rows_v, out_hbm.at[pl.ds(base, rows_per_w)])

    return gather_kernel(table, idx)


def _mm_body(tile_ids, expert_sel, lo_arr, hi_arr, xs_ref, w_ref, b_ref,
             out_ref):
    i = pl.program_id(0)
    t_cur = tile_ids[i]
    t_prev = tile_ids[jnp.maximum(i - 1, 0)]
    first_visit = jnp.logical_or(i == 0, t_cur != t_prev)
    lo = lo_arr[i]
    hi = hi_arr[i]

    @pl.when(first_visit)
    def _():
        out_ref[...] = jnp.zeros_like(out_ref)

    @pl.when(hi > lo)
    def _():
        rid = lax.broadcasted_iota(jnp.int32, (BLK, 1), 0)
        mask = (rid >= lo) & (rid < hi)
        acc = jnp.dot(xs_ref[...].astype(jnp.bfloat16),
                      w_ref[0].astype(jnp.bfloat16),
                      preferred_element_type=jnp.float32)
        out_ref[...] += jnp.where(mask, acc + b_ref[0], 0.0)


def _grouped_matmul(xs, w, b3, tile_ids, expert_sel, lo_rel, hi_rel):
    n, d = xs.shape
    num_items = tile_ids.shape[0]
    grid_spec = pltpu.PrefetchScalarGridSpec(
        num_scalar_prefetch=4,
        grid=(num_items,),
        in_specs=[
            pl.BlockSpec((BLK, d), lambda i, t, es, lo, hi: (t[i], 0)),
            pl.BlockSpec((1, d, d), lambda i, t, es, lo, hi: (es[i], 0, 0)),
            pl.BlockSpec((1, 1, d), lambda i, t, es, lo, hi: (es[i], 0, 0)),
        ],
        out_specs=pl.BlockSpec((BLK, d), lambda i, t, es, lo, hi: (t[i], 0)),
    )
    return pl.pallas_call(
        _mm_body,
        grid_spec=grid_spec,
        out_shape=jax.ShapeDtypeStruct((n, d), jnp.float32),
        compiler_params=pltpu.CompilerParams(
            dimension_semantics=("arbitrary",)),
    )(tile_ids, expert_sel, lo_rel, hi_rel, xs, w, b3)


def kernel(x, gate_idx, W, b):
    n, d = x.shape
    e_total = W.shape[0]
    num_tiles = n // BLK
    num_items = num_tiles + e_total - 1

    e_flat = gate_idx.reshape(n).astype(jnp.int32)
    xs, inv_perm, counts16, starts16 = _sc_route_and_scatter(
        x, e_flat, e_total)

    starts = starts16[:e_total]
    ends = starts + counts16[:e_total]
    tile_edges = jnp.arange(num_tiles, dtype=jnp.int32) * BLK
    bp = jnp.sort(jnp.concatenate([tile_edges, starts[1:]]))
    bp_next = jnp.concatenate([bp[1:], jnp.array([n], jnp.int32)])
    tile_ids = jnp.clip(bp // BLK, 0, num_tiles - 1).astype(jnp.int32)
    expert_sel = jnp.clip(
        jnp.sum((ends[None, :] <= bp[:, None]).astype(jnp.int32), axis=1),
        0, e_total - 1).astype(jnp.int32)
    lo_rel = (bp - tile_ids * BLK).astype(jnp.int32)
    hi_rel = (bp_next - tile_ids * BLK).astype(jnp.int32)

    b3 = b.reshape(e_total, 1, d)
    ys = _grouped_matmul(xs, W, b3, tile_ids, expert_sel, lo_rel, hi_rel)
    return _sc_row_gather(ys, inv_perm)


# speedup vs baseline: 1.3213x; 1.0688x over previous
import functools

import jax
import jax.numpy as jnp
from jax import lax
from jax.experimental import pallas as pl
from jax.experimental.pallas import tpu as pltpu
from jax.experimental.pallas import tpu_sc as plsc

BLK = 256

SC_CORES = 2
SC_SUBCORES = 16
SC_WORKERS = SC_CORES * SC_SUBCORES


def _sc_route_and_scatter(x, e_flat, n_expert, blk):
    n, d = x.shape
    per_w = n // SC_WORKERS
    n_vec = per_w // 16
    num_tiles = n // blk
    mesh = plsc.VectorSubcoreMesh(core_axis_name="c", subcore_axis_name="s")

    @functools.partial(
        pl.kernel,
        mesh=mesh,
        compiler_params=pltpu.CompilerParams(needs_layout_passes=False),
        out_type=(
            jax.ShapeDtypeStruct((n, d), x.dtype),
            jax.ShapeDtypeStruct((n,), jnp.int32),
            jax.ShapeDtypeStruct((16,), jnp.int32),
            jax.ShapeDtypeStruct((16,), jnp.int32),
            jax.ShapeDtypeStruct((16,), jnp.int32),
            jax.ShapeDtypeStruct((16,), jnp.int32),
            jax.ShapeDtypeStruct((SC_WORKERS, 16), jnp.int32),
        ),
        scratch_types=[
            pltpu.VMEM((per_w,), jnp.int32),
            pltpu.VMEM((per_w,), jnp.int32),
            pltpu.VMEM((per_w,), jnp.int32),
            pltpu.VMEM((per_w, d), x.dtype),
            pltpu.VMEM((16,), jnp.int32),
            pltpu.VMEM((16,), jnp.int32),
            pltpu.VMEM((SC_WORKERS, 16), jnp.int32),
            pltpu.SemaphoreType.DMA,
            pltpu.SemaphoreType.DMA,
        ],
    )
    def route_kernel(x_hbm, e_hbm, xs_hbm, invp_hbm, ti_hbm, es_hbm,
                     lo_hbm, hi_hbm, tbl_hbm, e_v, sib_v, dest_v, rows_v,
                     stage_v, stage2_v, tbl_v, sem_x, sem_sc):
        sid = lax.axis_index("s")
        cid = lax.axis_index("c")
        wid = sid * SC_CORES + cid
        base = wid * per_w
        lane = lax.iota(jnp.int32, 16)

        x_copy = pltpu.async_copy(x_hbm.at[pl.ds(base, per_w)], rows_v,
                                  sem_x)
        pltpu.sync_copy(e_hbm.at[pl.ds(base, per_w)], e_v)

        sib = sid * 2 + (1 - cid)
        pltpu.sync_copy(e_hbm.at[pl.ds(sib * per_w, per_w)], sib_v)
        for (src, row) in ((e_v, wid), (sib_v, sib)):
            vecs_h = [src[pl.ds(k * 16, 16)] for k in range(n_vec)]
            cnt_vec = jnp.zeros((16,), jnp.int32)
            for ex in range(n_expert):
                c = jnp.int32(0)
                for v in vecs_h:
                    c = c + jnp.sum((v == ex).astype(jnp.int32))
                cnt_vec = cnt_vec + jnp.where(lane == ex, c, 0)
            stage_v[...] = cnt_vec
            pltpu.sync_copy(stage_v, tbl_hbm.at[row])
        plsc.subcore_barrier()
        vecs = [e_v[pl.ds(k * 16, 16)] for k in range(n_vec)]

        pltpu.sync_copy(tbl_hbm, tbl_v)
        total_vec = jnp.zeros((16,), jnp.int32)
        before_vec = jnp.zeros((16,), jnp.int32)
        for r in range(SC_WORKERS):
            row = tbl_v[r]
            total_vec = total_vec + row
            before_vec = before_vec + row * jnp.where(
                jnp.int32(r) < wid, jnp.int32(1), jnp.int32(0))
        incl = jnp.cumsum(total_vec)
        starts_vec = incl - total_vec
        my_base_vec = starts_vec + before_vec

        @pl.when(wid == 0)
        def _():
            ends_vec = starts_vec + total_vec
            bp = jnp.full((16,), n, jnp.int32)
            hi = jnp.full((16,), n, jnp.int32)
            for t in range(num_tiles):
                v = jnp.int32(t * blk)
                m = (starts_vec < v) & (lane >= 1) & (lane < n_expert)
                r = plsc.all_reduce_population_count(m) + t
                bp = jnp.where(lane == r, v, bp)
                hi = jnp.where(lane == r - 1, v, hi)
            edges = lane * blk
            for j in range(1, n_expert):
                sv = starts_vec[j]
                m = (lane < num_tiles) & (edges <= sv)
                r = plsc.all_reduce_population_count(m) + (j - 1)
                bp = jnp.where(lane == r, sv, bp)
                hi = jnp.where(lane == r - 1, sv, hi)
            tiles = jnp.clip(bp // blk, 0, num_tiles - 1)
            esel = jnp.zeros((16,), jnp.int32)
            for ee in range(n_expert):
                endv = ends_vec[ee]
                esel = esel + jnp.where(endv <= bp, 1, 0)
            esel = jnp.clip(esel, 0, n_expert - 1)
            stage_v[...] = tiles
            pltpu.sync_copy(stage_v, ti_hbm)
            stage_v[...] = esel
            pltpu.sync_copy(stage_v, es_hbm)
            stage_v[...] = bp - tiles * blk
            pltpu.sync_copy(stage_v, lo_hbm)
            stage_v[...] = hi - tiles * blk
            pltpu.sync_copy(stage_v, hi_hbm)

        bases = [
            jnp.sum(jnp.where(lane == ex, my_base_vec, 0))
            for ex in range(n_expert)
        ]
        for k in range(n_vec):
            v = vecs[k]
            dest = jnp.zeros((16,), jnp.int32)
            for ex in range(n_expert):
                m = (v == ex).astype(jnp.int32)
                pc = jnp.cumsum(m)
                dest = dest + m * (bases[ex] + pc - 1)
                bases[ex] = bases[ex] + jnp.sum(m)
            dest_v[pl.ds(k * 16, 16)] = dest

        x_copy.wait()
        pltpu.async_copy(rows_v, xs_hbm.at[dest_v], sem_sc).wait()
        pltpu.sync_copy(dest_v, invp_hbm.at[pl.ds(base, per_w)])

    xs, invp, ti, es, lo, hi, _ = route_kernel(x, e_flat)
    return xs, invp, ti, es, lo, hi


def _sc_row_gather(table, idx):
    n_rows, d = table.shape
    b = idx.shape[0]
    rows_per_w = b // SC_WORKERS
    mesh = plsc.VectorSubcoreMesh(core_axis_name="c", subcore_axis_name="s")

    @functools.partial(
        pl.kernel,
        mesh=mesh,
        out_type=jax.ShapeDtypeStruct((b, d), table.dtype),
        scratch_types=[
            pltpu.VMEM((rows_per_w,), jnp.int32),
            pltpu.VMEM((rows_per_w, d), table.dtype),
            pltpu.SemaphoreType.DMA,
        ],
    )
    def gather_kernel(table_hbm, idx_hbm, out_hbm, idx_v, rows_v, sem):
        wid = lax.axis_index("s") * SC_CORES + lax.axis_index("c")
        base = wid * rows_per_w
        pltpu.sync_copy(idx_hbm.at[pl.ds(base, rows_per_w)], idx_v)
        pltpu.async_copy(table_hbm.at[idx_v], rows_v, sem).wait()
        pltpu.sync_copy(rows_v, out_hbm.at[pl.ds(base, rows_per_w)])

    return gather_kernel(table, idx)


def _mm_body(tile_ids, expert_sel, lo_arr, hi_arr, xs_ref, w_ref, b_ref,
             out_ref):
    i = pl.program_id(0)
    t_cur = tile_ids[i]
    t_prev = tile_ids[jnp.maximum(i - 1, 0)]
    first_visit = jnp.logical_or(i == 0, t_cur != t_prev)
    lo = lo_arr[i]
    hi = hi_arr[i]

    @pl.when(first_visit)
    def _():
        out_ref[...] = jnp.zeros_like(out_ref)

    @pl.when(hi > lo)
    def _():
        rid = lax.broadcasted_iota(jnp.int32, (BLK, 1), 0)
        mask = (rid >= lo) & (rid < hi)
        acc = jnp.dot(xs_ref[...].astype(jnp.bfloat16),
                      w_ref[0].astype(jnp.bfloat16),
                      preferred_element_type=jnp.float32)
        out_ref[...] += jnp.where(mask, acc + b_ref[0], 0.0)


def _grouped_matmul(xs, w, b3, tile_ids, expert_sel, lo_rel, hi_rel):
    n, d = xs.shape
    num_items = tile_ids.shape[0]
    grid_spec = pltpu.PrefetchScalarGridSpec(
        num_scalar_prefetch=4,
        grid=(num_items,),
        in_specs=[
            pl.BlockSpec((BLK, d), lambda i, t, es, lo, hi: (t[i], 0)),
            pl.BlockSpec((1, d, d), lambda i, t, es, lo, hi: (es[i], 0, 0)),
            pl.BlockSpec((1, 1, d), lambda i, t, es, lo, hi: (es[i], 0, 0)),
        ],
        out_specs=pl.BlockSpec((BLK, d), lambda i, t, es, lo, hi: (t[i], 0)),
    )
    return pl.pallas_call(
        _mm_body,
        grid_spec=grid_spec,
        out_shape=jax.ShapeDtypeStruct((n, d), jnp.float32),
        compiler_params=pltpu.CompilerParams(
            dimension_semantics=("arbitrary",)),
    )(tile_ids, expert_sel, lo_rel, hi_rel, xs, w, b3)


def kernel(x, gate_idx, W, b):
    n, d = x.shape
    e_total = W.shape[0]
    e_flat = gate_idx.reshape(n).astype(jnp.int32)
    xs, inv_perm, tile_ids, expert_sel, lo_rel, hi_rel = (
        _sc_route_and_scatter(x, e_flat, e_total, BLK))
    b3 = b.reshape(e_total, 1, d)
    ys = _grouped_matmul(xs, W, b3, tile_ids, expert_sel, lo_rel, hi_rel)
    return _sc_row_gather(ys, inv_perm)


# speedup vs baseline: 1.4045x; 1.0630x over previous
import functools

import jax
import jax.numpy as jnp
from jax import lax
from jax.experimental import pallas as pl
from jax.experimental.pallas import tpu as pltpu
from jax.experimental.pallas import tpu_sc as plsc

BLK = 512

SC_CORES = 2
SC_SUBCORES = 16
SC_WORKERS = SC_CORES * SC_SUBCORES


def _sc_route_and_scatter(x, e_flat, n_expert, blk):
    n, d = x.shape
    per_w = n // SC_WORKERS
    n_vec = per_w // 16
    num_tiles = n // blk
    mesh = plsc.VectorSubcoreMesh(core_axis_name="c", subcore_axis_name="s")

    @functools.partial(
        pl.kernel,
        mesh=mesh,
        compiler_params=pltpu.CompilerParams(needs_layout_passes=False),
        out_type=(
            jax.ShapeDtypeStruct((n, d), x.dtype),
            jax.ShapeDtypeStruct((n,), jnp.int32),
            jax.ShapeDtypeStruct((16,), jnp.int32),
            jax.ShapeDtypeStruct((16,), jnp.int32),
            jax.ShapeDtypeStruct((16,), jnp.int32),
            jax.ShapeDtypeStruct((16,), jnp.int32),
            jax.ShapeDtypeStruct((SC_WORKERS, 16), jnp.int32),
        ),
        scratch_types=[
            pltpu.VMEM((per_w,), jnp.int32),
            pltpu.VMEM((per_w,), jnp.int32),
            pltpu.VMEM((per_w,), jnp.int32),
            pltpu.VMEM((per_w, d), x.dtype),
            pltpu.VMEM((16,), jnp.int32),
            pltpu.VMEM((16,), jnp.int32),
            pltpu.VMEM((SC_WORKERS, 16), jnp.int32),
            pltpu.SemaphoreType.DMA,
            pltpu.SemaphoreType.DMA,
        ],
    )
    def route_kernel(x_hbm, e_hbm, xs_hbm, invp_hbm, ti_hbm, es_hbm,
                     lo_hbm, hi_hbm, tbl_hbm, e_v, sib_v, dest_v, rows_v,
                     stage_v, stage2_v, tbl_v, sem_x, sem_sc):
        sid = lax.axis_index("s")
        cid = lax.axis_index("c")
        wid = sid * SC_CORES + cid
        base = wid * per_w
        lane = lax.iota(jnp.int32, 16)

        x_copy = pltpu.async_copy(x_hbm.at[pl.ds(base, per_w)], rows_v,
                                  sem_x)
        pltpu.sync_copy(e_hbm.at[pl.ds(base, per_w)], e_v)

        sib = sid * 2 + (1 - cid)
        pltpu.sync_copy(e_hbm.at[pl.ds(sib * per_w, per_w)], sib_v)
        for (src, row) in ((e_v, wid), (sib_v, sib)):
            vecs_h = [src[pl.ds(k * 16, 16)] for k in range(n_vec)]
            cnt_vec = jnp.zeros((16,), jnp.int32)
            for ex in range(n_expert):
                c = jnp.int32(0)
                for v in vecs_h:
                    c = c + jnp.sum((v == ex).astype(jnp.int32))
                cnt_vec = cnt_vec + jnp.where(lane == ex, c, 0)
            stage_v[...] = cnt_vec
            pltpu.sync_copy(stage_v, tbl_hbm.at[row])
        plsc.subcore_barrier()
        vecs = [e_v[pl.ds(k * 16, 16)] for k in range(n_vec)]

        pltpu.sync_copy(tbl_hbm, tbl_v)
        total_vec = jnp.zeros((16,), jnp.int32)
        before_vec = jnp.zeros((16,), jnp.int32)
        for r in range(SC_WORKERS):
            row = tbl_v[r]
            total_vec = total_vec + row
            before_vec = before_vec + row * jnp.where(
                jnp.int32(r) < wid, jnp.int32(1), jnp.int32(0))
        incl = jnp.cumsum(total_vec)
        starts_vec = incl - total_vec
        my_base_vec = starts_vec + before_vec

        @pl.when(wid == 0)
        def _():
            ends_vec = starts_vec + total_vec
            bp = jnp.full((16,), n, jnp.int32)
            hi = jnp.full((16,), n, jnp.int32)
            for t in range(num_tiles):
                v = jnp.int32(t * blk)
                m = (starts_vec < v) & (lane >= 1) & (lane < n_expert)
                r = plsc.all_reduce_population_count(m) + t
                bp = jnp.where(lane == r, v, bp)
                hi = jnp.where(lane == r - 1, v, hi)
            edges = lane * blk
            for j in range(1, n_expert):
                sv = starts_vec[j]
                m = (lane < num_tiles) & (edges <= sv)
                r = plsc.all_reduce_population_count(m) + (j - 1)
                bp = jnp.where(lane == r, sv, bp)
                hi = jnp.where(lane == r - 1, sv, hi)
            tiles = jnp.clip(bp // blk, 0, num_tiles - 1)
            esel = jnp.zeros((16,), jnp.int32)
            for ee in range(n_expert):
                endv = ends_vec[ee]
                esel = esel + jnp.where(endv <= bp, 1, 0)
            esel = jnp.clip(esel, 0, n_expert - 1)
            stage_v[...] = tiles
            pltpu.sync_copy(stage_v, ti_hbm)
            stage_v[...] = esel
            pltpu.sync_copy(stage_v, es_hbm)
            stage_v[...] = bp - tiles * blk
            pltpu.sync_copy(stage_v, lo_hbm)
            stage_v[...] = hi - tiles * blk
            pltpu.sync_copy(stage_v, hi_hbm)

        bases = [
            jnp.sum(jnp.where(lane == ex, my_base_vec, 0))
            for ex in range(n_expert)
        ]
        for k in range(n_vec):
            v = vecs[k]
            dest = jnp.zeros((16,), jnp.int32)
            for ex in range(n_expert):
                m = (v == ex).astype(jnp.int32)
                pc = jnp.cumsum(m)
                dest = dest + m * (bases[ex] + pc - 1)
                bases[ex] = bases[ex] + jnp.sum(m)
            dest_v[pl.ds(k * 16, 16)] = dest

        x_copy.wait()
        pltpu.async_copy(rows_v, xs_hbm.at[dest_v], sem_sc).wait()
        pltpu.sync_copy(dest_v, invp_hbm.at[pl.ds(base, per_w)])

    xs, invp, ti, es, lo, hi, _ = route_kernel(x, e_flat)
    return xs, invp, ti, es, lo, hi


def _sc_row_gather(table, idx):
    n_rows, d = table.shape
    b = idx.shape[0]
    rows_per_w = b // SC_WORKERS
    mesh = plsc.VectorSubcoreMesh(core_axis_name="c", subcore_axis_name="s")

    @functools.partial(
        pl.kernel,
        mesh=mesh,
        out_type=jax.ShapeDtypeStruct((b, d), table.dtype),
        scratch_types=[
            pltpu.VMEM((rows_per_w,), jnp.int32),
            pltpu.VMEM((rows_per_w, d), table.dtype),
            pltpu.SemaphoreType.DMA,
        ],
    )
    def gather_kernel(table_hbm, idx_hbm, out_hbm, idx_v, rows_v, sem):
        wid = lax.axis_index("s") * SC_CORES + lax.axis_index("c")
        base = wid * rows_per_w
        pltpu.sync_copy(idx_hbm.at[pl.ds(base, rows_per_w)], idx_v)
        pltpu.async_copy(table_hbm.at[idx_v], rows_v, sem).wait()
        pltpu.sync_copy(rows_v, out_hbm.at[pl.ds(base, rows_per_w)])

    return gather_kernel(table, idx)


def _mm_body(tile_ids, expert_sel, lo_arr, hi_arr, xs_ref, w_ref, b_ref,
             out_ref):
    i = pl.program_id(0)
    t_cur = tile_ids[i]
    t_prev = tile_ids[jnp.maximum(i - 1, 0)]
    first_visit = jnp.logical_or(i == 0, t_cur != t_prev)
    lo = lo_arr[i]
    hi = hi_arr[i]

    @pl.when(first_visit)
    def _():
        out_ref[...] = jnp.zeros_like(out_ref)

    @pl.when(hi > lo)
    def _():
        rid = lax.broadcasted_iota(jnp.int32, (BLK, 1), 0)
        mask = (rid >= lo) & (rid < hi)
        acc = jnp.dot(xs_ref[...].astype(jnp.bfloat16),
                      w_ref[0].astype(jnp.bfloat16),
                      preferred_element_type=jnp.float32)
        out_ref[...] += jnp.where(mask, acc + b_ref[0], 0.0)


def _grouped_matmul(xs, w, b3, tile_ids, expert_sel, lo_rel, hi_rel):
    n, d = xs.shape
    num_items = tile_ids.shape[0]
    grid_spec = pltpu.PrefetchScalarGridSpec(
        num_scalar_prefetch=4,
        grid=(num_items,),
        in_specs=[
            pl.BlockSpec((BLK, d), lambda i, t, es, lo, hi: (t[i], 0)),
            pl.BlockSpec((1, d, d), lambda i, t, es, lo, hi: (es[i], 0, 0)),
            pl.BlockSpec((1, 1, d), lambda i, t, es, lo, hi: (es[i], 0, 0)),
        ],
        out_specs=pl.BlockSpec((BLK, d), lambda i, t, es, lo, hi: (t[i], 0)),
    )
    return pl.pallas_call(
        _mm_body,
        grid_spec=grid_spec,
        out_shape=jax.ShapeDtypeStruct((n, d), jnp.float32),
        compiler_params=pltpu.CompilerParams(
            dimension_semantics=("arbitrary",)),
    )(tile_ids, expert_sel, lo_rel, hi_rel, xs, w, b3)


def kernel(x, gate_idx, W, b):
    n, d = x.shape
    e_total = W.shape[0]
    e_flat = gate_idx.reshape(n).astype(jnp.int32)
    xs, inv_perm, tile_ids, expert_sel, lo_rel, hi_rel = (
        _sc_route_and_scatter(x, e_flat, e_total, BLK))
    b3 = b.reshape(e_total, 1, d)
    ys = _grouped_matmul(xs, W, b3, tile_ids, expert_sel, lo_rel, hi_rel)
    return _sc_row_gather(ys, inv_perm)
